# baseline (speedup 1.0000x reference)
import jax
import jax.numpy as jnp
from jax import lax
from jax.experimental import pallas as pl
from jax.experimental.pallas import tpu as pltpu
from jax.experimental.pallas import tpu_sc as plsc

N_CUR = 10000
D = 128
DH = D // 2
NP = 320000
NC, NS = 2, 16
NW = NC * NS
CH = 128
NCHUNK = NP // CH
KB = NCHUNK // NS
EXTRA = NCHUNK - KB * NS
KMAX = KB + 1
ZR = 632
ACC_ROWS = ZR * NS
ZRP = 656
HB = 2


def _sc_body(feat_hbm, idxf_hbm, seg_hbm, out_hbm, cnt_hbm,
             idxf_v, seg_v, rows0, rows1, rows2, rows3, cnt_v, hbuf, hacc,
             acc, gs0, gs1, gs2, gs3, ss0, ss1, ss2, ss3):
    c = lax.axis_index("c")
    s = lax.axis_index("s")
    g = c * NS + s
    has_extra = s < EXTRA

    pltpu.sync_copy(idxf_hbm.at[pl.ds(s * KB, KB)], idxf_v.at[pl.ds(0, KB)])
    pltpu.sync_copy(seg_hbm.at[pl.ds(s * KB, KB)], seg_v.at[pl.ds(0, KB)])

    @pl.when(has_extra)
    def _():
        pltpu.sync_copy(idxf_hbm.at[NS * KB + s], idxf_v.at[KB])
        pltpu.sync_copy(seg_hbm.at[NS * KB + s], seg_v.at[KB])

    def xform_body(i, _):
        for d in range(CH // 16):
            v = idxf_v[i, pl.ds(d * 16, 16)]
            idxf_v[i, pl.ds(d * 16, 16)] = v * 2 + c
        return 0

    lax.fori_loop(0, KMAX, xform_body, 0)

    zero16 = jnp.zeros((16,), jnp.float32)
    ones16 = jnp.ones((16,), jnp.float32)

    def zrows_body(i, _):
        for d in range(DH // 16):
            rows0[i, pl.ds(d * 16, 16)] = zero16
        return 0

    lax.fori_loop(0, CH, zrows_body, 0)

    def zcnt_body(i, _):
        cnt_v[pl.ds(i * 16, 16)] = zero16
        return 0

    lax.fori_loop(0, ACC_ROWS // 16, zcnt_body, 0)

    base = s * ZR
    for k in range(ZR // CH):
        pltpu.sync_copy(rows0, acc.at[pl.ds(base + k * CH, CH)])
    rem = ZR % CH
    if rem:
        off = base + (ZR // CH) * CH
        pltpu.sync_copy(rows0.at[pl.ds(0, rem)], acc.at[pl.ds(off, rem)])

    plsc.subcore_barrier()

    def fire(j, buf, sem):
        pltpu.async_copy(feat_hbm.at[idxf_v.at[j]], buf, sem)

    def drain(j, buf, sem):
        pltpu.make_async_copy(feat_hbm.at[idxf_v.at[j]], buf, sem).wait()

    def fire_s(j, buf, sem):
        pltpu.async_copy(buf, acc.at[seg_v.at[j]], sem, add=True)

    def wait_s(j, buf, sem):
        pltpu.make_async_copy(buf, acc.at[seg_v.at[j]], sem).wait()

    def count(j):
        for l in range(CH // 16):
            sv = seg_v[j, pl.ds(l * 16, 16)]
            plsc.addupdate_scatter(cnt_v, [sv], ones16)

    bufs = (rows0, rows1, rows2, rows3)
    sems = (gs0, gs1, gs2, gs3)
    ssems = (ss0, ss1, ss2, ss3)
    NB = len(bufs)

    def slot(j, b, do_wait=True, fire_next=True):
        drain(j, bufs[b], sems[b])
        fire_s(j, bufs[b], ssems[b])
        count(j)
        bp = (b - 1) % NB
        if do_wait:
            wait_s(j - 1, bufs[bp], ssems[bp])
        if fire_next:
            fire(j + NB - 1, bufs[bp], sems[bp])

    for b in range(NB - 1):
        fire(b, bufs[b], sems[b])
    slot(0, 0, do_wait=False)
    slot(1, 1)
    slot(2, 2)
    slot(3, 3)

    NQ_LO, NQ_HI = 1, (KB - 2 * NB + 1) // NB

    def quad_body(q, _):
        j0 = NB * q
        for b in range(NB):
            slot(j0 + b, b)
        return 0

    lax.fori_loop(NQ_LO, NQ_HI + 1, quad_body, 0)
    for j in range(NB * (NQ_HI + 1), KB):
        slot(j, j % NB, fire_next=(j + NB - 1 < KB))
    wait_s(KB - 1, bufs[(KB - 1) % NB], ssems[(KB - 1) % NB])

    @pl.when(has_extra)
    def _():
        fire(KB, rows0, gs0)
        drain(KB, rows0, gs0)
        fire_s(KB, rows0, ss0)
        count(KB)
        wait_s(KB, rows0, ss0)

    pltpu.sync_copy(cnt_v, cnt_hbm.at[g])
    plsc.subcore_barrier()

    def zh_body(i, _):
        hacc[pl.ds(i * 16, 16)] = zero16
        for t in range(HB):
            hbuf[t, pl.ds(i * 16, 16)] = zero16
        return 0

    lax.fori_loop(0, ZRP // 16, zh_body, 0)

    for t4 in range(NS // HB):
        pltpu.sync_copy(
            cnt_hbm.at[pl.ds(c * NS + t4 * HB, HB), pl.ds(base, ZR)],
            hbuf.at[pl.ds(0, HB), pl.ds(0, ZR)])

        def hadd_body(i, _):
            v = hacc[pl.ds(i * 16, 16)]
            for t in range(HB):
                v = v + hbuf[t, pl.ds(i * 16, 16)]
            hacc[pl.ds(i * 16, 16)] = v
            return 0

        lax.fori_loop(0, ZRP // 16, hadd_body, 0)

    def hinv_body(i, _):
        v = hacc[pl.ds(i * 16, 16)]
        hacc[pl.ds(i * 16, 16)] = 1.0 / jnp.maximum(v, 1.0)
        return 0

    lax.fori_loop(0, ZRP // 16, hinv_body, 0)

    def do_block(r0, n):
        pltpu.sync_copy(acc.at[pl.ds(base + r0, n)], rows0.at[pl.ds(0, n)])

        def mrow(i, _):
            iv = hacc[pl.ds(r0 + i, 16)][0]
            for d in range(DH // 16):
                rows0[i, pl.ds(d * 16, 16)] = rows0[i, pl.ds(d * 16, 16)] * iv
            return 0

        lax.fori_loop(0, n, mrow, 0)

        @pl.when(c == 0)
        def _():
            pltpu.sync_copy(rows0.at[pl.ds(0, n)],
                            out_hbm.at[pl.ds(base + r0, n), pl.ds(0, DH)])

        @pl.when(c == 1)
        def _():
            pltpu.sync_copy(rows0.at[pl.ds(0, n)],
                            out_hbm.at[pl.ds(base + r0, n), pl.ds(DH, DH)])

    for k in range(ZR // CH):
        do_block(k * CH, CH)

    @pl.when(s < NS - 1)
    def _():
        do_block((ZR // CH) * CH, ZR % CH)

    @pl.when(s == NS - 1)
    def _():
        do_block((ZR // CH) * CH, N_CUR - (NS - 1) * ZR - (ZR // CH) * CH)


_sc_call = pl.kernel(
    _sc_body,
    out_type=[
        jax.ShapeDtypeStruct((N_CUR, D), jnp.float32),
        jax.ShapeDtypeStruct((NW, ACC_ROWS), jnp.float32),
    ],
    mesh=plsc.VectorSubcoreMesh(
        core_axis_name="c", subcore_axis_name="s",
        num_cores=NC, num_subcores=NS),
    compiler_params=pltpu.CompilerParams(
        use_tc_tiling_on_sc=False, needs_layout_passes=False),
    scratch_types=[
        pltpu.VMEM((KMAX, CH), jnp.int32),
        pltpu.VMEM((KMAX, CH), jnp.int32),
        pltpu.VMEM((CH, DH), jnp.float32),
        pltpu.VMEM((CH, DH), jnp.float32),
        pltpu.VMEM((CH, DH), jnp.float32),
        pltpu.VMEM((CH, DH), jnp.float32),
        pltpu.VMEM((ACC_ROWS,), jnp.float32),
        pltpu.VMEM((HB, ZRP), jnp.float32),
        pltpu.VMEM((ZRP,), jnp.float32),
        pltpu.VMEM_SHARED((ACC_ROWS, DH), jnp.float32),
        pltpu.SemaphoreType.DMA,
        pltpu.SemaphoreType.DMA,
        pltpu.SemaphoreType.DMA,
        pltpu.SemaphoreType.DMA,
        pltpu.SemaphoreType.DMA,
        pltpu.SemaphoreType.DMA,
        pltpu.SemaphoreType.DMA,
        pltpu.SemaphoreType.DMA,
    ],
)


@jax.jit
def kernel(features, coors_inv_last, coors_inv, coors):
    del coors
    feat_h = features.reshape(-1, DH)
    idxf = coors_inv_last.astype(jnp.int32).reshape(NCHUNK, CH)
    seg = coors_inv.astype(jnp.int32).reshape(NCHUNK, CH)
    out, _ = _sc_call(feat_h, idxf, seg)
    return out
